# baseline (speedup 1.0000x reference)
import functools

import jax
import jax.numpy as jnp
from jax import lax
from jax.experimental import pallas as pl
from jax.experimental.pallas import tpu as pltpu
from jax.experimental.pallas import tpu_sc as plsc

_ROWS = 16384
_ATOMS = 200
_LANES = 16
_VPR = -(-_ATOMS // _LANES)
_TAIL = _ATOMS - (_VPR - 1) * _LANES

_info = plsc.get_sparse_core_info()
_NC, _NS = _info.num_cores, _info.num_subcores
_NW = _NC * _NS
_RPW = _ROWS // _NW
_CH = 128
_NCHUNK = _RPW // _CH
_BUF_WORDS = _CH * _ATOMS + _LANES


def _body(sp_hbm, en_hbm, tab_hbm, out_hbm,
          tab_v, en_v, out_v, buf0, buf1, sem0, sem1):
    wid = lax.axis_index("s") * _NC + lax.axis_index("c")
    base = wid * _RPW
    pltpu.sync_copy(tab_hbm, tab_v)
    pltpu.sync_copy(en_hbm.at[pl.ds(base, _RPW)], en_v)
    bufs = (buf0, buf1)
    sems = (sem0, sem1)

    def start(c):
        src = sp_hbm.at[pl.ds((base + c * _CH) * _ATOMS, _CH * _ATOMS)]
        dst = bufs[c % 2].at[pl.ds(0, _CH * _ATOMS)]
        return pltpu.async_copy(src, dst, sems[c % 2])

    cur = start(0)
    tv = tab_v[...]
    lane = lax.iota(jnp.int32, _LANES)
    lane_off = lane * _ATOMS
    zero = jnp.zeros((_LANES,), jnp.float32)

    def lookup(s):
        return lax.gather(
            tv, (s & 15)[:, None],
            dimension_numbers=lax.GatherDimensionNumbers(
                offset_dims=(), collapsed_slice_dims=(0,),
                start_index_map=(0,)),
            slice_sizes=(1,),
            mode=lax.GatherScatterMode.PROMISE_IN_BOUNDS)

    for c in range(_NCHUNK):
        cur.wait()
        if c + 1 < _NCHUNK:
            nxt = start(c + 1)
        buf = bufs[c % 2]

        for g in range(_CH // _LANES):
            idx0 = lane_off + g * _LANES * _ATOMS

            def step(i, carry):
                i0, a0, a1, a2, a3 = carry
                s0 = plsc.load_gather(buf, [i0])
                s1 = plsc.load_gather(buf, [i0 + 1])
                s2 = plsc.load_gather(buf, [i0 + 2])
                s3 = plsc.load_gather(buf, [i0 + 3])
                return (i0 + 4, a0 + lookup(s0), a1 + lookup(s1),
                        a2 + lookup(s2), a3 + lookup(s3))

            _, a0, a1, a2, a3 = lax.fori_loop(
                0, _ATOMS // 4, step, (idx0, zero, zero, zero, zero))
            out_v[pl.ds(c * _CH + g * _LANES, _LANES)] = (a0 + a1) + (a2 + a3)

        if c + 1 < _NCHUNK:
            cur = nxt

    for i in range(_RPW // _LANES):
        sl = pl.ds(i * _LANES, _LANES)
        out_v[sl] = out_v[sl] + en_v[sl]
    pltpu.sync_copy(out_v, out_hbm.at[pl.ds(base, _RPW)])


_sc_call = functools.partial(
    pl.kernel,
    mesh=plsc.VectorSubcoreMesh(core_axis_name="c", subcore_axis_name="s"),
    compiler_params=pltpu.CompilerParams(needs_layout_passes=False),
    out_type=jax.ShapeDtypeStruct((_ROWS,), jnp.float32),
    scratch_types=[
        pltpu.VMEM((_LANES,), jnp.float32),
        pltpu.VMEM((_RPW,), jnp.float32),
        pltpu.VMEM((_RPW,), jnp.float32),
        pltpu.VMEM((_BUF_WORDS,), jnp.int32),
        pltpu.VMEM((_BUF_WORDS,), jnp.int32),
        pltpu.SemaphoreType.DMA,
        pltpu.SemaphoreType.DMA,
    ],
)(_body)


def kernel(species, energies, self_energies):
    sp_flat = species.reshape(-1).astype(jnp.int32)
    tab16 = (jnp.zeros((_LANES,), jnp.float32)
             .at[: self_energies.shape[0]].set(self_energies.astype(jnp.float32)))
    out = _sc_call(sp_flat, energies.astype(jnp.float32), tab16)
    return (species, out)


# speedup vs baseline: 1.0191x; 1.0191x over previous
import functools

import jax
import jax.numpy as jnp
from jax import lax
from jax.experimental import pallas as pl
from jax.experimental.pallas import tpu as pltpu
from jax.experimental.pallas import tpu_sc as plsc

_ROWS = 16384
_ATOMS = 200
_LANES = 16
_VPR = -(-_ATOMS // _LANES)
_TAIL = _ATOMS - (_VPR - 1) * _LANES

_info = plsc.get_sparse_core_info()
_NC, _NS = _info.num_cores, _info.num_subcores
_NW = _NC * _NS
_RPW = _ROWS // _NW
_CH = 128
_NCHUNK = _RPW // _CH
_BUF_WORDS = _CH * _ATOMS + _LANES


def _body(sp_hbm, en_hbm, tab_hbm, out_hbm,
          tab_v, en_v, out_v, buf0, buf1, sem0, sem1):
    wid = lax.axis_index("s") * _NC + lax.axis_index("c")
    base = wid * _RPW
    pltpu.sync_copy(tab_hbm, tab_v)
    pltpu.sync_copy(en_hbm.at[pl.ds(base, _RPW)], en_v)
    bufs = (buf0, buf1)
    sems = (sem0, sem1)

    def start(c):
        src = sp_hbm.at[pl.ds((base + c * _CH) * _ATOMS, _CH * _ATOMS)]
        dst = bufs[c % 2].at[pl.ds(0, _CH * _ATOMS)]
        return pltpu.async_copy(src, dst, sems[c % 2])

    cur = start(0)
    tv = tab_v[...]
    lane = lax.iota(jnp.int32, _LANES)
    lane_off = lane * _ATOMS
    zero = jnp.zeros((_LANES,), jnp.float32)

    def lookup(s):
        return lax.gather(
            tv, (s & 15)[:, None],
            dimension_numbers=lax.GatherDimensionNumbers(
                offset_dims=(), collapsed_slice_dims=(0,),
                start_index_map=(0,)),
            slice_sizes=(1,),
            mode=lax.GatherScatterMode.PROMISE_IN_BOUNDS)

    for c in range(_NCHUNK):
        cur.wait()
        if c + 1 < _NCHUNK:
            nxt = start(c + 1)
        buf = bufs[c % 2]

        def group(g, carry):
            idx0 = lane_off + g * (_LANES * _ATOMS)

            def step(i, c2):
                i0, a0, a1, a2, a3 = c2
                s0 = plsc.load_gather(buf, [i0])
                s1 = plsc.load_gather(buf, [i0 + 1])
                s2 = plsc.load_gather(buf, [i0 + 2])
                s3 = plsc.load_gather(buf, [i0 + 3])
                return (i0 + 4, a0 + lookup(s0), a1 + lookup(s1),
                        a2 + lookup(s2), a3 + lookup(s3))

            _, a0, a1, a2, a3 = lax.fori_loop(
                0, _ATOMS // 4, step, (idx0, zero, zero, zero, zero),
                unroll=10)
            off = pl.multiple_of(c * _CH + g * _LANES, _LANES)
            out_v[pl.ds(off, _LANES)] = (a0 + a1) + (a2 + a3)
            return carry

        lax.fori_loop(0, _CH // _LANES, group, 0)

        if c + 1 < _NCHUNK:
            cur = nxt

    for i in range(_RPW // _LANES):
        sl = pl.ds(i * _LANES, _LANES)
        out_v[sl] = out_v[sl] + en_v[sl]
    pltpu.sync_copy(out_v, out_hbm.at[pl.ds(base, _RPW)])


_sc_call = functools.partial(
    pl.kernel,
    mesh=plsc.VectorSubcoreMesh(core_axis_name="c", subcore_axis_name="s"),
    compiler_params=pltpu.CompilerParams(needs_layout_passes=False),
    out_type=jax.ShapeDtypeStruct((_ROWS,), jnp.float32),
    scratch_types=[
        pltpu.VMEM((_LANES,), jnp.float32),
        pltpu.VMEM((_RPW,), jnp.float32),
        pltpu.VMEM((_RPW,), jnp.float32),
        pltpu.VMEM((_BUF_WORDS,), jnp.int32),
        pltpu.VMEM((_BUF_WORDS,), jnp.int32),
        pltpu.SemaphoreType.DMA,
        pltpu.SemaphoreType.DMA,
    ],
)(_body)


def kernel(species, energies, self_energies):
    sp_flat = species.reshape(-1).astype(jnp.int32)
    tab16 = (jnp.zeros((_LANES,), jnp.float32)
             .at[: self_energies.shape[0]].set(self_energies.astype(jnp.float32)))
    out = _sc_call(sp_flat, energies.astype(jnp.float32), tab16)
    return (species, out)


# speedup vs baseline: 1.1933x; 1.1710x over previous
import functools

import jax
import jax.numpy as jnp
from jax import lax
from jax.experimental import pallas as pl
from jax.experimental.pallas import tpu as pltpu
from jax.experimental.pallas import tpu_sc as plsc

_ROWS = 16384
_ATOMS = 200
_LANES = 16

_info = plsc.get_sparse_core_info()
_NC, _NS = _info.num_cores, _info.num_subcores
_NW = _NC * _NS
_RPW = _ROWS // _NW
_CH = 64
_NCHUNK = _RPW // _CH
_NBUF = 4
_GRP = _CH // _LANES
_PLAIN = 176


def _body(sp_hbm, en_hbm, tab_hbm, spo_hbm, eno_hbm,
          tab_v, en_v, out_v,
          b0, b1, b2, b3, si0, si1, si2, si3, so0, so1, so2, so3):
    wid = lax.axis_index("s") * _NC + lax.axis_index("c")
    base = wid * _RPW
    pltpu.sync_copy(tab_hbm, tab_v)
    pltpu.sync_copy(en_hbm.at[pl.ds(base, _RPW)], en_v)
    bufs = (b0, b1, b2, b3)
    isems = (si0, si1, si2, si3)
    osems = (so0, so1, so2, so3)

    def start_in(c):
        return pltpu.async_copy(
            sp_hbm.at[pl.ds(base + c * _CH, _CH), :], bufs[c % _NBUF],
            isems[c % _NBUF])

    def start_out(c):
        return pltpu.async_copy(
            bufs[c % _NBUF], spo_hbm.at[pl.ds(base + c * _CH, _CH), :],
            osems[c % _NBUF])

    tv = tab_v[...]
    lane = lax.iota(jnp.int32, _LANES)
    start_col = (lane * 5) & 15
    zero = jnp.zeros((_LANES,), jnp.float32)

    def lookup(s):
        return lax.gather(
            tv, (s & 15)[:, None],
            dimension_numbers=lax.GatherDimensionNumbers(
                offset_dims=(), collapsed_slice_dims=(0,),
                start_index_map=(0,)),
            slice_sizes=(1,),
            mode=lax.GatherScatterMode.PROMISE_IN_BOUNDS)

    in_cps = [start_in(c) for c in range(_NBUF)]
    out_cps = [None] * _NCHUNK

    for c in range(_NCHUNK):
        if c >= 1 and c + _NBUF - 1 < _NCHUNK:
            out_cps[c - 1].wait()
            in_cps.append(start_in(c + _NBUF - 1))
        in_cps[c].wait()
        buf = bufs[c % _NBUF]

        def group(g, carry):
            rvec = lane + g * _LANES

            def stepA(i, c2):
                col, a0, a1, a2, a3 = c2
                v0 = plsc.load_gather(buf, [rvec, col])
                v1 = plsc.load_gather(buf, [rvec, col + 1])
                v2 = plsc.load_gather(buf, [rvec, col + 2])
                v3 = plsc.load_gather(buf, [rvec, col + 3])
                return (col + 4, a0 + lookup(v0), a1 + lookup(v1),
                        a2 + lookup(v2), a3 + lookup(v3))

            col, a0, a1, a2, a3 = lax.fori_loop(
                0, _PLAIN // 4, stepA, (start_col, zero, zero, zero, zero),
                unroll=11)
            accs = [a0, a1, a2, a3]
            for t in range(_PLAIN, _ATOMS):
                cs = start_col + t
                if t + 15 >= _ATOMS:
                    cs = jnp.where(cs >= _ATOMS, cs - _ATOMS, cs)
                accs[t % 4] = accs[t % 4] + lookup(
                    plsc.load_gather(buf, [rvec, cs]))
            off = pl.multiple_of(c * _CH + g * _LANES, _LANES)
            out_v[pl.ds(off, _LANES)] = ((accs[0] + accs[1]) +
                                         (accs[2] + accs[3]))
            return carry

        lax.fori_loop(0, _GRP, group, 0)
        out_cps[c] = start_out(c)

    for c in range(_NCHUNK - _NBUF + 1, _NCHUNK):
        out_cps[c].wait()

    for i in range(_RPW // _LANES):
        sl = pl.ds(i * _LANES, _LANES)
        out_v[sl] = out_v[sl] + en_v[sl]
    pltpu.sync_copy(out_v, eno_hbm.at[pl.ds(base, _RPW)])


_sc_call = functools.partial(
    pl.kernel,
    mesh=plsc.VectorSubcoreMesh(core_axis_name="c", subcore_axis_name="s"),
    compiler_params=pltpu.CompilerParams(needs_layout_passes=False),
    out_type=(
        jax.ShapeDtypeStruct((_ROWS, _ATOMS), jnp.int32),
        jax.ShapeDtypeStruct((_ROWS,), jnp.float32),
    ),
    scratch_types=[
        pltpu.VMEM((_LANES,), jnp.float32),
        pltpu.VMEM((_RPW,), jnp.float32),
        pltpu.VMEM((_RPW,), jnp.float32),
        pltpu.VMEM((_CH, _ATOMS), jnp.int32),
        pltpu.VMEM((_CH, _ATOMS), jnp.int32),
        pltpu.VMEM((_CH, _ATOMS), jnp.int32),
        pltpu.VMEM((_CH, _ATOMS), jnp.int32),
        pltpu.SemaphoreType.DMA,
        pltpu.SemaphoreType.DMA,
        pltpu.SemaphoreType.DMA,
        pltpu.SemaphoreType.DMA,
        pltpu.SemaphoreType.DMA,
        pltpu.SemaphoreType.DMA,
        pltpu.SemaphoreType.DMA,
        pltpu.SemaphoreType.DMA,
    ],
)(_body)


def kernel(species, energies, self_energies):
    sp = jnp.asarray(species, jnp.int32)
    tab16 = (jnp.zeros((_LANES,), jnp.float32)
             .at[: self_energies.shape[0]].set(self_energies.astype(jnp.float32)))
    sp_out, en_out = _sc_call(sp, energies.astype(jnp.float32), tab16)
    return (sp_out, en_out)


# speedup vs baseline: 2.2817x; 1.9121x over previous
import functools

import jax
import jax.numpy as jnp
from jax import lax
from jax.experimental import pallas as pl
from jax.experimental.pallas import tpu as pltpu
from jax.experimental.pallas import tpu_sc as plsc

_ROWS = 16384
_ATOMS = 200
_LANES = 16

_info = plsc.get_sparse_core_info()
_NC, _NS = _info.num_cores, _info.num_subcores
_NW = _NC * _NS
_RPW = _ROWS // _NW
_CH = 128
_NCHUNK = _RPW // _CH
_GRP = _CH // _LANES


def _body(sp_hbm, en_hbm, tab_hbm, spo_hbm, eno_hbm,
          tab_v, en_v, out_v,
          b0, b1, b2, b3, si0, si1, si2, si3, so0, so1, so2, so3):
    wid = lax.axis_index("s") * _NC + lax.axis_index("c")
    base = wid * _RPW
    pltpu.sync_copy(tab_hbm, tab_v)
    pltpu.sync_copy(en_hbm.at[pl.ds(base, _RPW)], en_v)
    bufs = (b0, b1, b2, b3)
    isems = (si0, si1, si2, si3)
    osems = (so0, so1, so2, so3)

    in_cps = [
        pltpu.async_copy(
            sp_hbm.at[:, pl.ds(base + c * _CH, _CH)], bufs[c], isems[c])
        for c in range(_NCHUNK)
    ]

    tv = tab_v[...]
    zero = jnp.zeros((_LANES,), jnp.float32)

    def lookup(s):
        return lax.gather(
            tv, (s & 15)[:, None],
            dimension_numbers=lax.GatherDimensionNumbers(
                offset_dims=(), collapsed_slice_dims=(0,),
                start_index_map=(0,)),
            slice_sizes=(1,),
            mode=lax.GatherScatterMode.PROMISE_IN_BOUNDS)

    out_cps = []
    for c in range(_NCHUNK):
        in_cps[c].wait()
        buf = bufs[c]

        def group(g, carry):
            c0 = pl.multiple_of(g * _LANES, _LANES)
            sl = pl.ds(c0, _LANES)

            def step(i, c2):
                a0, a1, a2, a3 = c2
                r = i * 4
                return (a0 + lookup(buf[r, sl]),
                        a1 + lookup(buf[r + 1, sl]),
                        a2 + lookup(buf[r + 2, sl]),
                        a3 + lookup(buf[r + 3, sl]))

            a0, a1, a2, a3 = lax.fori_loop(
                0, _ATOMS // 4, step, (zero, zero, zero, zero), unroll=10)
            off = pl.multiple_of(c * _CH + g * _LANES, _LANES)
            out_v[pl.ds(off, _LANES)] = (a0 + a1) + (a2 + a3)
            return carry

        lax.fori_loop(0, _GRP, group, 0)
        out_cps.append(pltpu.async_copy(
            buf, spo_hbm.at[:, pl.ds(base + c * _CH, _CH)], osems[c]))

    for i in range(_RPW // _LANES):
        sl = pl.ds(i * _LANES, _LANES)
        out_v[sl] = out_v[sl] + en_v[sl]
    pltpu.sync_copy(out_v, eno_hbm.at[pl.ds(base, _RPW)])
    for cp in out_cps:
        cp.wait()


_sc_call = functools.partial(
    pl.kernel,
    mesh=plsc.VectorSubcoreMesh(core_axis_name="c", subcore_axis_name="s"),
    compiler_params=pltpu.CompilerParams(needs_layout_passes=False),
    out_type=(
        jax.ShapeDtypeStruct((_ATOMS, _ROWS), jnp.int32),
        jax.ShapeDtypeStruct((_ROWS,), jnp.float32),
    ),
    scratch_types=[
        pltpu.VMEM((_LANES,), jnp.float32),
        pltpu.VMEM((_RPW,), jnp.float32),
        pltpu.VMEM((_RPW,), jnp.float32),
        pltpu.VMEM((_ATOMS, _CH), jnp.int32),
        pltpu.VMEM((_ATOMS, _CH), jnp.int32),
        pltpu.VMEM((_ATOMS, _CH), jnp.int32),
        pltpu.VMEM((_ATOMS, _CH), jnp.int32),
        pltpu.SemaphoreType.DMA,
        pltpu.SemaphoreType.DMA,
        pltpu.SemaphoreType.DMA,
        pltpu.SemaphoreType.DMA,
        pltpu.SemaphoreType.DMA,
        pltpu.SemaphoreType.DMA,
        pltpu.SemaphoreType.DMA,
        pltpu.SemaphoreType.DMA,
    ],
)(_body)


def kernel(species, energies, self_energies):
    spt = jnp.asarray(species, jnp.int32).T
    tab16 = (jnp.zeros((_LANES,), jnp.float32)
             .at[: self_energies.shape[0]].set(self_energies.astype(jnp.float32)))
    spo_t, en_out = _sc_call(spt, energies.astype(jnp.float32), tab16)
    return (spo_t.T, en_out)


# speedup vs baseline: 2.3462x; 1.0283x over previous
import functools

import jax
import jax.numpy as jnp
from jax import lax
from jax.experimental import pallas as pl
from jax.experimental.pallas import tpu as pltpu
from jax.experimental.pallas import tpu_sc as plsc

_ROWS = 16384
_ATOMS = 200
_LANES = 16

_info = plsc.get_sparse_core_info()
_NC, _NS = _info.num_cores, _info.num_subcores
_NW = _NC * _NS
_RPW = _ROWS // _NW
_CH = 128
_NCHUNK = _RPW // _CH
_GRP = _CH // _LANES


def _body(sp_hbm, en_hbm, tab_hbm, spo_hbm, eno_hbm,
          tab_v, en_v, out_v,
          b0, b1, b2, b3, si0, si1, si2, si3, so0, so1, so2, so3):
    wid = lax.axis_index("s") * _NC + lax.axis_index("c")
    base = wid * _RPW
    pltpu.sync_copy(tab_hbm, tab_v.at[pl.ds(0, 7)])
    pltpu.sync_copy(en_hbm.at[pl.ds(base, _RPW)], en_v)
    bufs = (b0, b1, b2, b3)
    isems = (si0, si1, si2, si3)
    osems = (so0, so1, so2, so3)

    in_cps = [
        pltpu.async_copy(
            sp_hbm.at[:, pl.ds(base + c * _CH, _CH)], bufs[c], isems[c])
        for c in range(_NCHUNK)
    ]

    lane = lax.iota(jnp.int32, _LANES)
    tv = jnp.where(lane < 7, tab_v[...], 0.0)
    zero = jnp.zeros((_LANES,), jnp.float32)

    def lookup(s):
        return lax.gather(
            tv, (s & 15)[:, None],
            dimension_numbers=lax.GatherDimensionNumbers(
                offset_dims=(), collapsed_slice_dims=(0,),
                start_index_map=(0,)),
            slice_sizes=(1,),
            mode=lax.GatherScatterMode.PROMISE_IN_BOUNDS)

    out_cps = []
    for c in range(_NCHUNK):
        in_cps[c].wait()
        buf = bufs[c]

        def group(g, carry):
            c0 = pl.multiple_of(g * _LANES, _LANES)
            sl = pl.ds(c0, _LANES)

            def step(i, c2):
                a0, a1, a2, a3 = c2
                r = i * 4
                return (a0 + lookup(buf[r, sl]),
                        a1 + lookup(buf[r + 1, sl]),
                        a2 + lookup(buf[r + 2, sl]),
                        a3 + lookup(buf[r + 3, sl]))

            a0, a1, a2, a3 = lax.fori_loop(
                0, _ATOMS // 4, step, (zero, zero, zero, zero), unroll=10)
            off = pl.multiple_of(c * _CH + g * _LANES, _LANES)
            out_v[pl.ds(off, _LANES)] = (a0 + a1) + (a2 + a3)
            return carry

        lax.fori_loop(0, _GRP, group, 0)
        out_cps.append(pltpu.async_copy(
            buf, spo_hbm.at[:, pl.ds(base + c * _CH, _CH)], osems[c]))

    for i in range(_RPW // _LANES):
        sl = pl.ds(i * _LANES, _LANES)
        out_v[sl] = out_v[sl] + en_v[sl]
    pltpu.sync_copy(out_v, eno_hbm.at[pl.ds(base, _RPW)])
    for cp in out_cps:
        cp.wait()


_sc_call = functools.partial(
    pl.kernel,
    mesh=plsc.VectorSubcoreMesh(core_axis_name="c", subcore_axis_name="s"),
    compiler_params=pltpu.CompilerParams(
        needs_layout_passes=False, skip_device_barrier=True),
    out_type=(
        jax.ShapeDtypeStruct((_ATOMS, _ROWS), jnp.int32),
        jax.ShapeDtypeStruct((_ROWS,), jnp.float32),
    ),
    scratch_types=[
        pltpu.VMEM((_LANES,), jnp.float32),
        pltpu.VMEM((_RPW,), jnp.float32),
        pltpu.VMEM((_RPW,), jnp.float32),
        pltpu.VMEM((_ATOMS, _CH), jnp.int32),
        pltpu.VMEM((_ATOMS, _CH), jnp.int32),
        pltpu.VMEM((_ATOMS, _CH), jnp.int32),
        pltpu.VMEM((_ATOMS, _CH), jnp.int32),
        pltpu.SemaphoreType.DMA,
        pltpu.SemaphoreType.DMA,
        pltpu.SemaphoreType.DMA,
        pltpu.SemaphoreType.DMA,
        pltpu.SemaphoreType.DMA,
        pltpu.SemaphoreType.DMA,
        pltpu.SemaphoreType.DMA,
        pltpu.SemaphoreType.DMA,
    ],
)(_body)


def kernel(species, energies, self_energies):
    spt = jnp.asarray(species, jnp.int32).T
    spo_t, en_out = _sc_call(spt, energies.astype(jnp.float32),
                             self_energies.astype(jnp.float32))
    return (spo_t.T, en_out)


# speedup vs baseline: 2.4328x; 1.0369x over previous
import functools

import jax
import jax.numpy as jnp
from jax import lax
from jax.experimental import pallas as pl
from jax.experimental.pallas import tpu as pltpu
from jax.experimental.pallas import tpu_sc as plsc

_ROWS = 16384
_ATOMS = 200
_LANES = 16

_info = plsc.get_sparse_core_info()
_NC, _NS = _info.num_cores, _info.num_subcores
_NW = _NC * _NS
_RPW = _ROWS // _NW
_CH = 128
_NCHUNK = _RPW // _CH
_GRP = _CH // _LANES


def _body(sp_hbm, en_hbm, tab_hbm, spo_hbm, eno_hbm,
          tab_v, en_v, out_v,
          b0, b1, b2, b3, si0, si1, si2, si3, so0, so1, so2, so3):
    wid = lax.axis_index("s") * _NC + lax.axis_index("c")
    base = wid * _RPW
    bufs = (b0, b1, b2, b3)
    isems = (si0, si1, si2, si3)
    osems = (so0, so1, so2, so3)

    in_cps = [
        pltpu.async_copy(
            sp_hbm.at[:, pl.ds(base + c * _CH, _CH)], bufs[c], isems[c])
        for c in range(_NCHUNK)
    ]
    pltpu.sync_copy(tab_hbm, tab_v.at[pl.ds(0, 7)])
    pltpu.sync_copy(en_hbm.at[pl.ds(base, _RPW)], en_v)

    lane = lax.iota(jnp.int32, _LANES)
    tv = jnp.where(lane < 7, tab_v[...], 0.0)
    zero = jnp.zeros((_LANES,), jnp.float32)

    def lookup(s):
        return lax.gather(
            tv, (s & 15)[:, None],
            dimension_numbers=lax.GatherDimensionNumbers(
                offset_dims=(), collapsed_slice_dims=(0,),
                start_index_map=(0,)),
            slice_sizes=(1,),
            mode=lax.GatherScatterMode.PROMISE_IN_BOUNDS)

    out_cps = []
    for c in range(_NCHUNK):
        in_cps[c].wait()
        buf = bufs[c]
        out_cps.append(pltpu.async_copy(
            buf, spo_hbm.at[:, pl.ds(base + c * _CH, _CH)], osems[c]))

        def group(g, carry):
            c0 = pl.multiple_of(g * _LANES, _LANES)
            sl = pl.ds(c0, _LANES)

            def step(i, c2):
                a0, a1, a2, a3 = c2
                r = i * 4
                return (a0 + lookup(buf[r, sl]),
                        a1 + lookup(buf[r + 1, sl]),
                        a2 + lookup(buf[r + 2, sl]),
                        a3 + lookup(buf[r + 3, sl]))

            a0, a1, a2, a3 = lax.fori_loop(
                0, _ATOMS // 4, step, (zero, zero, zero, zero), unroll=10)
            off = pl.multiple_of(c * _CH + g * _LANES, _LANES)
            out_v[pl.ds(off, _LANES)] = (a0 + a1) + (a2 + a3)
            return carry

        lax.fori_loop(0, _GRP, group, 0)

    for i in range(_RPW // _LANES):
        sl = pl.ds(i * _LANES, _LANES)
        out_v[sl] = out_v[sl] + en_v[sl]
    pltpu.sync_copy(out_v, eno_hbm.at[pl.ds(base, _RPW)])
    for cp in out_cps:
        cp.wait()


_sc_call = functools.partial(
    pl.kernel,
    mesh=plsc.VectorSubcoreMesh(core_axis_name="c", subcore_axis_name="s"),
    compiler_params=pltpu.CompilerParams(
        needs_layout_passes=False, skip_device_barrier=True),
    out_type=(
        jax.ShapeDtypeStruct((_ATOMS, _ROWS), jnp.int32),
        jax.ShapeDtypeStruct((_ROWS,), jnp.float32),
    ),
    scratch_types=[
        pltpu.VMEM((_LANES,), jnp.float32),
        pltpu.VMEM((_RPW,), jnp.float32),
        pltpu.VMEM((_RPW,), jnp.float32),
        pltpu.VMEM((_ATOMS, _CH), jnp.int32),
        pltpu.VMEM((_ATOMS, _CH), jnp.int32),
        pltpu.VMEM((_ATOMS, _CH), jnp.int32),
        pltpu.VMEM((_ATOMS, _CH), jnp.int32),
        pltpu.SemaphoreType.DMA,
        pltpu.SemaphoreType.DMA,
        pltpu.SemaphoreType.DMA,
        pltpu.SemaphoreType.DMA,
        pltpu.SemaphoreType.DMA,
        pltpu.SemaphoreType.DMA,
        pltpu.SemaphoreType.DMA,
        pltpu.SemaphoreType.DMA,
    ],
)(_body)


def kernel(species, energies, self_energies):
    spt = jnp.asarray(species, jnp.int32).T
    spo_t, en_out = _sc_call(spt, energies.astype(jnp.float32),
                             self_energies.astype(jnp.float32))
    return (spo_t.T, en_out)
